# baseline (speedup 1.0000x reference)
import functools

import jax
import jax.numpy as jnp
from jax import lax
from jax.experimental import pallas as pl
from jax.experimental.pallas import tpu as pltpu
from jax.experimental.pallas import tpu_sc as plsc

TOPIC_K = 128
HALF_K = TOPIC_K // 2
NROWS_TBL = 100000
BATCH = 4096
DEG1 = 32
DEG2 = 64
LANES = 16
SEGW = HALF_K // LANES

NC = 2
NS = 16
NW = NC * NS
PER_W = BATCH // NW
CH = 2
NCH = PER_W // CH
NBUF = 4


def _pack_body(x_ref, out_ref):
    u = lax.bitcast_convert_type(x_ref[...], jnp.uint32)
    r = u + jnp.uint32(0x7FFF) + ((u >> 16) & jnp.uint32(1))
    hi16 = (r >> 16).astype(jnp.int32)
    lo = hi16[:, :HALF_K]
    hi = hi16[:, HALF_K:]
    out_ref[...] = lo | (hi << 16)


def _pack_table(tbl):
    rb = 2000
    return pl.pallas_call(
        _pack_body,
        grid=(NROWS_TBL // rb,),
        in_specs=[pl.BlockSpec((rb, TOPIC_K), lambda i: (i, 0))],
        out_specs=pl.BlockSpec((rb, HALF_K), lambda i: (i, 0)),
        out_shape=jax.ShapeDtypeStruct((NROWS_TBL, HALF_K), jnp.int32),
    )(tbl)


def _softmax_rows_accumulate(rows_ref, row0, nrows, ninv, acc_ref, out_row):
    ninv_vec = jnp.full((LANES,), ninv, jnp.float32)
    zeros = tuple(jnp.zeros((LANES,), jnp.float32) for _ in range(2 * SEGW))

    @plsc.parallel_loop(0, nrows, carry=zeros)
    def accs(r, accs):
        evs = [None] * (2 * SEGW)
        tot = None
        for s in range(SEGW):
            w = rows_ref[row0 + r, pl.ds(s * LANES, LANES)]
            bf = plsc.bitcast(w, jnp.bfloat16)
            a, b = plsc.unpack(bf, format=plsc.PackFormat.INTERLEAVED)
            ea = jnp.exp(a)
            eb = jnp.exp(b)
            evs[s] = ea
            evs[SEGW + s] = eb
            contrib = ea + eb
            tot = contrib if tot is None else tot + contrib
        ssum = jnp.sum(tot)
        scale = ninv_vec / lax.broadcast(ssum, (LANES,))
        return tuple(a + e * scale for a, e in zip(accs, evs))

    for s in range(2 * SEGW):
        acc_ref[out_row, pl.ds(s * LANES, LANES)] = accs[s]


def _sc_body(v_hbm, h1_hbm, h2_hbm, edoc_hbm, eword_hbm,
             selfout_hbm, agg1_hbm, agg2_hbm,
             idx_self, idx1, idx2, self_rows, buf1, buf2, acc1, acc2,
             sem_self, sem1_0, sem1_1, sem1_2, sem1_3,
             sem2_0, sem2_1, sem2_2, sem2_3):
    cid = lax.axis_index("c")
    sid = lax.axis_index("s")
    wid = sid * NC + cid
    base = wid * PER_W

    pltpu.sync_copy(v_hbm.at[pl.ds(base, PER_W)], idx_self)
    pltpu.sync_copy(h1_hbm.at[pl.ds(base * DEG1, PER_W * DEG1)], idx1)
    pltpu.sync_copy(h2_hbm.at[pl.ds(base * DEG2, PER_W * DEG2)], idx2)

    pltpu.async_copy(edoc_hbm.at[idx_self], self_rows, sem_self)

    sems1 = (sem1_0, sem1_1, sem1_2, sem1_3)
    sems2 = (sem2_0, sem2_1, sem2_2, sem2_3)

    def chunk_slices(c):
        return (idx1.at[pl.ds(c * CH * DEG1, CH * DEG1)],
                idx2.at[pl.ds(c * CH * DEG2, CH * DEG2)])

    def start_chunk(c, b):
        s1, s2 = chunk_slices(c)
        pltpu.async_copy(eword_hbm.at[s1], buf1.at[b], sems1[b])
        pltpu.async_copy(edoc_hbm.at[s2], buf2.at[b], sems2[b])

    def wait_chunk(c, b):
        s1, s2 = chunk_slices(c)
        pltpu.make_async_copy(eword_hbm.at[s1], buf1.at[b], sems1[b]).wait()
        pltpu.make_async_copy(edoc_hbm.at[s2], buf2.at[b], sems2[b]).wait()

    for b in range(NBUF):
        start_chunk(b, b)

    def outer(i, carry):
        for b in range(NBUF):
            c = NBUF * i + b
            wait_chunk(c, b)
            for j in range(CH):
                e = c * CH + j
                _softmax_rows_accumulate(buf1.at[b], j * DEG1, DEG1,
                                         1.0 / DEG1, acc1, e)
                _softmax_rows_accumulate(buf2.at[b], j * DEG2, DEG2,
                                         1.0 / DEG2, acc2, e)
            nc = c + NBUF

            @pl.when(nc < NCH)
            def _():
                start_chunk(nc, b)
        return carry

    lax.fori_loop(0, NCH // NBUF, outer, 0)

    pltpu.make_async_copy(edoc_hbm.at[idx_self], self_rows, sem_self).wait()
    pltpu.sync_copy(self_rows, selfout_hbm.at[pl.ds(base, PER_W)])
    pltpu.sync_copy(acc1, agg1_hbm.at[pl.ds(base, PER_W)])
    pltpu.sync_copy(acc2, agg2_hbm.at[pl.ds(base, PER_W)])


@functools.cache
def _build_sc_gather():
    mesh = plsc.VectorSubcoreMesh(core_axis_name="c", subcore_axis_name="s")
    aggout = jax.ShapeDtypeStruct((BATCH, TOPIC_K), jnp.float32)
    selfout = jax.ShapeDtypeStruct((BATCH, HALF_K), jnp.int32)
    return pl.kernel(
        _sc_body,
        mesh=mesh,
        compiler_params=pltpu.CompilerParams(needs_layout_passes=False,
                                             use_tc_tiling_on_sc=False),
        out_type=(selfout, aggout, aggout),
        scratch_types=(
            pltpu.VMEM((PER_W,), jnp.int32),
            pltpu.VMEM((PER_W * DEG1,), jnp.int32),
            pltpu.VMEM((PER_W * DEG2,), jnp.int32),
            pltpu.VMEM((PER_W, HALF_K), jnp.int32),
            pltpu.VMEM((NBUF, CH * DEG1, HALF_K), jnp.int32),
            pltpu.VMEM((NBUF, CH * DEG2, HALF_K), jnp.int32),
            pltpu.VMEM((PER_W, TOPIC_K), jnp.float32),
            pltpu.VMEM((PER_W, TOPIC_K), jnp.float32),
        ) + (pltpu.SemaphoreType.DMA,) * (1 + 2 * NBUF),
    )


def _tc_body(selfw_ref, a1_ref, a2_ref, ws_ref, w1_ref, w2_ref, out_ref):
    w = selfw_ref[...]
    lo = lax.bitcast_convert_type(w << 16, jnp.float32)
    hi = lax.bitcast_convert_type(w & jnp.int32(-65536), jnp.float32)
    x = jnp.concatenate([lo, hi], axis=1)
    x = x - jnp.max(x, axis=-1, keepdims=True)
    e = jnp.exp(x)
    h = e / jnp.sum(e, axis=-1, keepdims=True)
    acc = jnp.dot(h, ws_ref[...], preferred_element_type=jnp.float32)
    acc = acc + jnp.dot(a1_ref[...], w1_ref[...],
                        preferred_element_type=jnp.float32)
    acc = acc + jnp.dot(a2_ref[...], w2_ref[...],
                        preferred_element_type=jnp.float32)
    acc = acc - jnp.max(acc, axis=-1, keepdims=True)
    ee = jnp.exp(acc)
    out_ref[...] = ee / jnp.sum(ee, axis=-1, keepdims=True)


def _tc_combine(self_packed, a1, a2, ws, w1, w2):
    bb = 512
    row_spec = pl.BlockSpec((bb, TOPIC_K), lambda i: (i, 0))
    self_spec = pl.BlockSpec((bb, HALF_K), lambda i: (i, 0))
    w_spec = pl.BlockSpec((TOPIC_K, TOPIC_K), lambda i: (0, 0))
    return pl.pallas_call(
        _tc_body,
        grid=(BATCH // bb,),
        in_specs=[self_spec, row_spec, row_spec, w_spec, w_spec, w_spec],
        out_specs=row_spec,
        out_shape=jax.ShapeDtypeStruct((BATCH, TOPIC_K), jnp.float32),
    )(self_packed, a1, a2, ws, w1, w2)


def kernel(v, one_hop_list, two_hop_list, E_doc, E_word,
           W_self, W_hop1, W_hop2):
    v32 = v.astype(jnp.int32)
    h1 = one_hop_list.reshape(-1).astype(jnp.int32)
    h2 = two_hop_list.reshape(-1).astype(jnp.int32)
    edoc_p = _pack_table(E_doc)
    eword_p = _pack_table(E_word)
    self_packed, agg1, agg2 = _build_sc_gather()(v32, h1, h2, edoc_p, eword_p)
    return _tc_combine(self_packed, agg1, agg2, W_self, W_hop1, W_hop2)


# speedup vs baseline: 2.3534x; 2.3534x over previous
import functools

import jax
import jax.numpy as jnp
from jax import lax
from jax.experimental import pallas as pl
from jax.experimental.pallas import tpu as pltpu
from jax.experimental.pallas import tpu_sc as plsc

TOPIC_K = 128
BATCH = 4096
DEG1 = 32
DEG2 = 64
LANES = 16
SEG = TOPIC_K // LANES

NC = 2
NS = 16
NW = NC * NS
PER_W = BATCH // NW
CH = 1
NCH = PER_W // CH
NBUF = 4


def _softmax_rows_accumulate(rows_ref, row0, nrows, ninv, acc_ref, out_row):
    ninv_vec = jnp.full((LANES,), ninv, jnp.float32)
    zeros = tuple(jnp.zeros((LANES,), jnp.float32) for _ in range(SEG))

    @plsc.parallel_loop(0, nrows, carry=zeros)
    def accs(r, accs):
        evs = []
        tot = None
        for s in range(SEG):
            x = rows_ref[row0 + r, pl.ds(s * LANES, LANES)]
            e = jnp.exp(x)
            evs.append(e)
            tot = e if tot is None else tot + e
        ssum = jnp.sum(tot)
        scale = ninv_vec / lax.broadcast(ssum, (LANES,))
        return tuple(a + e * scale for a, e in zip(accs, evs))
    for s in range(SEG):
        acc_ref[out_row, pl.ds(s * LANES, LANES)] = accs[s]


def _sc_body(v_hbm, h1_hbm, h2_hbm, edoc_hbm, eword_hbm,
             selfout_hbm, agg1_hbm, agg2_hbm,
             idx_self, idx1, idx2, self_rows, buf1, buf2, acc1, acc2,
             sem_self, sem1_0, sem1_1, sem1_2, sem1_3,
             sem2_0, sem2_1, sem2_2, sem2_3):
    cid = lax.axis_index("c")
    sid = lax.axis_index("s")
    wid = sid * NC + cid
    base = wid * PER_W

    pltpu.sync_copy(v_hbm.at[pl.ds(base, PER_W)], idx_self)
    pltpu.sync_copy(h1_hbm.at[pl.ds(base * DEG1, PER_W * DEG1)], idx1)
    pltpu.sync_copy(h2_hbm.at[pl.ds(base * DEG2, PER_W * DEG2)], idx2)

    pltpu.async_copy(edoc_hbm.at[idx_self], self_rows, sem_self)

    sems1 = (sem1_0, sem1_1, sem1_2, sem1_3)
    sems2 = (sem2_0, sem2_1, sem2_2, sem2_3)

    def chunk_slices(c):
        return (idx1.at[pl.ds(c * CH * DEG1, CH * DEG1)],
                idx2.at[pl.ds(c * CH * DEG2, CH * DEG2)])

    def start_chunk(c, b):
        s1, s2 = chunk_slices(c)
        pltpu.async_copy(eword_hbm.at[s1], buf1.at[b], sems1[b])
        pltpu.async_copy(edoc_hbm.at[s2], buf2.at[b], sems2[b])

    def wait_chunk(c, b):
        s1, s2 = chunk_slices(c)
        pltpu.make_async_copy(eword_hbm.at[s1], buf1.at[b], sems1[b]).wait()
        pltpu.make_async_copy(edoc_hbm.at[s2], buf2.at[b], sems2[b]).wait()

    for b in range(NBUF):
        start_chunk(b, b)

    def outer(i, carry):
        for b in range(NBUF):
            c = NBUF * i + b
            wait_chunk(c, b)
            for j in range(CH):
                e = c * CH + j
                _softmax_rows_accumulate(buf1.at[b], j * DEG1, DEG1,
                                         1.0 / DEG1, acc1, e)
                _softmax_rows_accumulate(buf2.at[b], j * DEG2, DEG2,
                                         1.0 / DEG2, acc2, e)
            nc = c + NBUF

            @pl.when(nc < NCH)
            def _():
                start_chunk(nc, b)
        return carry

    lax.fori_loop(0, NCH // NBUF, outer, 0)

    pltpu.make_async_copy(edoc_hbm.at[idx_self], self_rows, sem_self).wait()
    pltpu.sync_copy(self_rows, selfout_hbm.at[pl.ds(base, PER_W)])
    pltpu.sync_copy(acc1, agg1_hbm.at[pl.ds(base, PER_W)])
    pltpu.sync_copy(acc2, agg2_hbm.at[pl.ds(base, PER_W)])


@functools.cache
def _build_sc_gather():
    mesh = plsc.VectorSubcoreMesh(core_axis_name="c", subcore_axis_name="s")
    out = jax.ShapeDtypeStruct((BATCH, TOPIC_K), jnp.float32)
    return pl.kernel(
        _sc_body,
        mesh=mesh,
        compiler_params=pltpu.CompilerParams(needs_layout_passes=False),
        out_type=(out, out, out),
        scratch_types=(
            pltpu.VMEM((PER_W,), jnp.int32),
            pltpu.VMEM((PER_W * DEG1,), jnp.int32),
            pltpu.VMEM((PER_W * DEG2,), jnp.int32),
            pltpu.VMEM((PER_W, TOPIC_K), jnp.float32),
            pltpu.VMEM((NBUF, CH * DEG1, TOPIC_K), jnp.float32),
            pltpu.VMEM((NBUF, CH * DEG2, TOPIC_K), jnp.float32),
            pltpu.VMEM((PER_W, TOPIC_K), jnp.float32),
            pltpu.VMEM((PER_W, TOPIC_K), jnp.float32),
        ) + (pltpu.SemaphoreType.DMA,) * (1 + 2 * NBUF),
    )


def _tc_body(self_ref, a1_ref, a2_ref, ws_ref, w1_ref, w2_ref, out_ref):
    x = self_ref[...]
    x = x - jnp.max(x, axis=-1, keepdims=True)
    e = jnp.exp(x)
    h = e / jnp.sum(e, axis=-1, keepdims=True)
    acc = jnp.dot(h, ws_ref[...], preferred_element_type=jnp.float32)
    acc = acc + jnp.dot(a1_ref[...], w1_ref[...],
                        preferred_element_type=jnp.float32)
    acc = acc + jnp.dot(a2_ref[...], w2_ref[...],
                        preferred_element_type=jnp.float32)
    acc = acc - jnp.max(acc, axis=-1, keepdims=True)
    ee = jnp.exp(acc)
    out_ref[...] = ee / jnp.sum(ee, axis=-1, keepdims=True)


def _tc_combine(self_raw, a1, a2, ws, w1, w2):
    bb = 512
    row_spec = pl.BlockSpec((bb, TOPIC_K), lambda i: (i, 0))
    w_spec = pl.BlockSpec((TOPIC_K, TOPIC_K), lambda i: (0, 0))
    return pl.pallas_call(
        _tc_body,
        grid=(BATCH // bb,),
        in_specs=[row_spec, row_spec, row_spec, w_spec, w_spec, w_spec],
        out_specs=row_spec,
        out_shape=jax.ShapeDtypeStruct((BATCH, TOPIC_K), jnp.float32),
    )(self_raw, a1, a2, ws, w1, w2)


def kernel(v, one_hop_list, two_hop_list, E_doc, E_word,
           W_self, W_hop1, W_hop2):
    v32 = v.astype(jnp.int32)
    h1 = one_hop_list.reshape(-1).astype(jnp.int32)
    h2 = two_hop_list.reshape(-1).astype(jnp.int32)
    self_raw, agg1, agg2 = _build_sc_gather()(v32, h1, h2, E_doc, E_word)
    return _tc_combine(self_raw, agg1, agg2, W_self, W_hop1, W_hop2)
